# Initial kernel scaffold; baseline (speedup 1.0000x reference)
import jax
import jax.numpy as jnp
from jax.experimental import pallas as pl


def kernel(token_idxs, table):
    raise NotImplementedError("write your pallas kernel here")


# speedup vs baseline: 3.3262x; 3.3262x over previous
import functools

import jax
import jax.numpy as jnp
from jax import lax
from jax.experimental import pallas as pl
from jax.experimental.pallas import tpu as pltpu
from jax.experimental.pallas import tpu_sc as plsc

_CHUNK = 128
_NC = 2
_NS = 16
_NW = _NC * _NS


@functools.lru_cache(maxsize=None)
def _build(n_chunks, d):
    n_w = n_chunks // _NW
    mesh = plsc.VectorSubcoreMesh(core_axis_name="c", subcore_axis_name="s")

    @functools.partial(
        pl.kernel,
        mesh=mesh,
        out_type=jax.ShapeDtypeStruct((n_chunks * _CHUNK, d), jnp.float32),
        scratch_types=[
            pltpu.VMEM((n_w, _CHUNK), jnp.int32),
            pltpu.VMEM((2, _CHUNK, d), jnp.float32),
            pltpu.SemaphoreType.DMA((2,)),
        ],
    )
    def emb(idx_hbm, table_hbm, out_hbm, idx_v, rows_v, gsem):
        wid = lax.axis_index("s") * _NC + lax.axis_index("c")
        crow = wid * n_w
        pltpu.sync_copy(idx_hbm.at[wid], idx_v)
        pltpu.make_async_copy(
            table_hbm.at[idx_v.at[0]], rows_v.at[0], gsem.at[0]
        ).start()

        def step(j, carry):
            @pl.when(j + 1 < n_w)
            def _():
                pltpu.make_async_copy(
                    table_hbm.at[idx_v.at[j + 1]],
                    rows_v.at[(j + 1) % 2],
                    gsem.at[(j + 1) % 2],
                ).start()

            pltpu.make_async_copy(
                table_hbm.at[idx_v.at[j]], rows_v.at[j % 2], gsem.at[j % 2]
            ).wait()
            pltpu.sync_copy(
                rows_v.at[j % 2], out_hbm.at[pl.ds((crow + j) * _CHUNK, _CHUNK)]
            )
            return carry

        lax.fori_loop(0, n_w, step, 0)

    return emb


def kernel(token_idxs, table):
    b, l = token_idxs.shape
    v, d = table.shape
    n = b * l
    span = _NW * _CHUNK
    n_pad = ((n + span - 1) // span) * span
    idx_flat = token_idxs.reshape(-1).astype(jnp.int32)
    if n_pad != n:
        idx_flat = jnp.concatenate(
            [idx_flat, jnp.zeros((n_pad - n,), jnp.int32)]
        )
    idx3d = idx_flat.reshape(_NW, n_pad // span, _CHUNK)
    out = _build(n_pad // _CHUNK, d)(idx3d, table)
    if n_pad != n:
        out = out[:n]
    return out.reshape(1, b, l, d)


# speedup vs baseline: 3.3321x; 1.0018x over previous
import functools

import jax
import jax.numpy as jnp
from jax import lax
from jax.experimental import pallas as pl
from jax.experimental.pallas import tpu as pltpu
from jax.experimental.pallas import tpu_sc as plsc

_CHUNK = 128
_NC = 2
_NS = 16
_NW = _NC * _NS
_NBUF = 4
_INFLIGHT = 2


@functools.lru_cache(maxsize=None)
def _build(n_chunks, d):
    n_w = n_chunks // _NW
    mesh = plsc.VectorSubcoreMesh(core_axis_name="c", subcore_axis_name="s")

    @functools.partial(
        pl.kernel,
        mesh=mesh,
        out_type=jax.ShapeDtypeStruct((n_chunks * _CHUNK, d), jnp.float32),
        scratch_types=[
            pltpu.VMEM((n_w, _CHUNK), jnp.int32),
            pltpu.VMEM((_NBUF, _CHUNK, d), jnp.float32),
            pltpu.SemaphoreType.DMA((_NBUF,)),
            pltpu.SemaphoreType.DMA((_NBUF,)),
        ],
    )
    def emb(idx_hbm, table_hbm, out_hbm, idx_v, rows_v, gsem, ssem):
        wid = lax.axis_index("s") * _NC + lax.axis_index("c")
        crow = wid * n_w
        pltpu.sync_copy(idx_hbm.at[wid], idx_v)

        def gather(j):
            b = j % _NBUF
            pltpu.make_async_copy(
                table_hbm.at[idx_v.at[j]], rows_v.at[b], gsem.at[b]
            ).start()

        def gather_wait(j):
            b = j % _NBUF
            pltpu.make_async_copy(
                table_hbm.at[idx_v.at[j]], rows_v.at[b], gsem.at[b]
            ).wait()

        def store(j):
            b = j % _NBUF
            pltpu.make_async_copy(
                rows_v.at[b], out_hbm.at[pl.ds((crow + j) * _CHUNK, _CHUNK)],
                ssem.at[b],
            ).start()

        def store_wait(j):
            b = j % _NBUF
            pltpu.make_async_copy(
                rows_v.at[b], out_hbm.at[pl.ds((crow + j) * _CHUNK, _CHUNK)],
                ssem.at[b],
            ).wait()

        for j in range(_INFLIGHT):
            gather(j)

        def step(j, carry):
            gather_wait(j)
            @pl.when(j >= _NBUF - _INFLIGHT)
            def _():
                store_wait(j - (_NBUF - _INFLIGHT))

            store(j)

            @pl.when(j + _INFLIGHT < n_w)
            def _():
                gather(j + _INFLIGHT)

            return carry

        lax.fori_loop(0, n_w, step, 0)
        for j in range(n_w - (_NBUF - _INFLIGHT), n_w):
            store_wait(j)

    return emb


def kernel(token_idxs, table):
    b, l = token_idxs.shape
    v, d = table.shape
    n = b * l
    span = _NW * _CHUNK
    n_pad = ((n + span - 1) // span) * span
    idx_flat = token_idxs.reshape(-1).astype(jnp.int32)
    if n_pad != n:
        idx_flat = jnp.concatenate(
            [idx_flat, jnp.zeros((n_pad - n,), jnp.int32)]
        )
    idx3d = idx_flat.reshape(_NW, n_pad // span, _CHUNK)
    out = _build(n_pad // _CHUNK, d)(idx3d, table)
    if n_pad != n:
        out = out[:n]
    return out.reshape(1, b, l, d)


# speedup vs baseline: 5.2686x; 1.5811x over previous
import functools

import jax
import jax.numpy as jnp
from jax import lax
from jax.experimental import pallas as pl
from jax.experimental.pallas import tpu as pltpu
from jax.experimental.pallas import tpu_sc as plsc

_NC = 2
_NS = 16
_NW = _NC * _NS
_NSEQ = 4
_NBUF = 3
_INFLIGHT = 2


@functools.lru_cache(maxsize=None)
def _build(b, l, d):
    seq_w = b // _NW
    t_w = seq_w // _NSEQ
    mesh = plsc.VectorSubcoreMesh(core_axis_name="c", subcore_axis_name="s")

    @functools.partial(
        pl.kernel,
        mesh=mesh,
        out_type=jax.ShapeDtypeStruct((b, l, d), jnp.float32),
        scratch_types=[
            pltpu.VMEM((seq_w, l), jnp.int32),
            pltpu.VMEM((_NBUF, _NSEQ, l, d), jnp.float32),
            pltpu.SemaphoreType.DMA((_NBUF,)),
            pltpu.SemaphoreType.DMA((_NBUF,)),
        ],
    )
    def emb(idx_hbm, table_hbm, out_hbm, idx_v, rows_v, gsem, ssem):
        wid = lax.axis_index("s") * _NC + lax.axis_index("c")
        sbase = wid * seq_w
        pltpu.sync_copy(idx_hbm.at[wid], idx_v)

        def gathers(t, start):
            bb = t % _NBUF
            for k in range(_NSEQ):
                cp = pltpu.make_async_copy(
                    table_hbm.at[idx_v.at[t * _NSEQ + k]],
                    rows_v.at[bb, k],
                    gsem.at[bb],
                )
                cp.start() if start else cp.wait()

        def store(t, start):
            bb = t % _NBUF
            cp = pltpu.make_async_copy(
                rows_v.at[bb],
                out_hbm.at[pl.ds(sbase + t * _NSEQ, _NSEQ)],
                ssem.at[bb],
            )
            cp.start() if start else cp.wait()

        for t in range(_INFLIGHT):
            gathers(t, start=True)

        def step(t, carry):
            gathers(t, start=False)
            @pl.when(t >= _NBUF - _INFLIGHT)
            def _():
                store(t - (_NBUF - _INFLIGHT), start=False)

            store(t, start=True)

            @pl.when(t + _INFLIGHT < t_w)
            def _():
                gathers(t + _INFLIGHT, start=True)

            return carry

        lax.fori_loop(0, t_w, step, 0)
        for t in range(max(0, t_w - (_NBUF - _INFLIGHT)), t_w):
            store(t, start=False)

    return emb


def kernel(token_idxs, table):
    b, l = token_idxs.shape
    v, d = table.shape
    span = _NW * _NSEQ
    b_pad = ((b + span - 1) // span) * span
    idx = token_idxs.astype(jnp.int32)
    if b_pad != b:
        idx = jnp.concatenate([idx, jnp.zeros((b_pad - b, l), jnp.int32)])
    idx3d = idx.reshape(_NW, b_pad // _NW, l)
    out = _build(b_pad, l, d)(idx3d, table)
    if b_pad != b:
        out = out[:b]
    return out.reshape(1, b, l, d)


# speedup vs baseline: 5.9610x; 1.1314x over previous
import functools

import jax
import jax.numpy as jnp
from jax import lax
from jax.experimental import pallas as pl
from jax.experimental.pallas import tpu as pltpu
from jax.experimental.pallas import tpu_sc as plsc

_NC = 2
_NS = 16
_NW = _NC * _NS
_NSEQ = 4
_NBUF = 3
_INFLIGHT = 2


@functools.lru_cache(maxsize=None)
def _build(b, l, d):
    seq_w = b // _NW
    t_w = seq_w // _NSEQ
    mesh = plsc.VectorSubcoreMesh(core_axis_name="c", subcore_axis_name="s")

    @functools.partial(
        pl.kernel,
        mesh=mesh,
        out_type=jax.ShapeDtypeStruct((1, b, l, d), jnp.float32),
        scratch_types=[
            pltpu.VMEM((seq_w, l), jnp.int32),
            pltpu.VMEM((_NBUF, _NSEQ, l, d), jnp.float32),
            pltpu.SemaphoreType.DMA((_NBUF,)),
            pltpu.SemaphoreType.DMA((_NBUF,)),
        ],
    )
    def emb(idx_hbm, table_hbm, out_hbm, idx_v, rows_v, gsem, ssem):
        wid = lax.axis_index("s") * _NC + lax.axis_index("c")
        sbase = wid * seq_w
        pltpu.sync_copy(idx_hbm.at[pl.ds(sbase, seq_w)], idx_v)

        def gathers(t, start):
            bb = t % _NBUF
            for k in range(_NSEQ):
                cp = pltpu.make_async_copy(
                    table_hbm.at[idx_v.at[t * _NSEQ + k]],
                    rows_v.at[bb, k],
                    gsem.at[bb],
                )
                cp.start() if start else cp.wait()

        def store(t, start):
            bb = t % _NBUF
            cp = pltpu.make_async_copy(
                rows_v.at[bb],
                out_hbm.at[0, pl.ds(sbase + t * _NSEQ, _NSEQ)],
                ssem.at[bb],
            )
            cp.start() if start else cp.wait()

        for t in range(_INFLIGHT):
            gathers(t, start=True)

        def step(t, carry):
            gathers(t, start=False)
            @pl.when(t >= _NBUF - _INFLIGHT)
            def _():
                store(t - (_NBUF - _INFLIGHT), start=False)

            store(t, start=True)

            @pl.when(t + _INFLIGHT < t_w)
            def _():
                gathers(t + _INFLIGHT, start=True)

            return carry

        lax.fori_loop(0, t_w, step, 0)
        for t in range(max(0, t_w - (_NBUF - _INFLIGHT)), t_w):
            store(t, start=False)

    return emb


def kernel(token_idxs, table):
    b, l = token_idxs.shape
    v, d = table.shape
    return _build(b, l, d)(token_idxs.astype(jnp.int32), table)


# speedup vs baseline: 10.7303x; 1.8001x over previous
import functools

import jax
import jax.numpy as jnp
from jax import lax
from jax.experimental import pallas as pl
from jax.experimental.pallas import tpu as pltpu
from jax.experimental.pallas import tpu_sc as plsc

_NC = 2
_NS = 16
_NW = _NC * _NS
_SEQ = 128
_NBUF = 3
_INFLIGHT = 2


@functools.lru_cache(maxsize=None)
def _build(b, l, d):
    mesh = plsc.VectorSubcoreMesh(core_axis_name="c", subcore_axis_name="s")

    @functools.partial(
        pl.kernel,
        mesh=mesh,
        out_type=jax.ShapeDtypeStruct((l, b, d), jnp.float32),
        scratch_types=[
            pltpu.VMEM((l, _SEQ), jnp.int32),
            pltpu.VMEM((_NBUF, _SEQ, d), jnp.float32),
            pltpu.SemaphoreType.DMA((_NBUF,)),
            pltpu.SemaphoreType.DMA((_NBUF,)),
        ],
    )
    def emb(idx_hbm, table_hbm, out_hbm, idx_v, rows_v, gsem, ssem):
        wid = lax.axis_index("s") * _NC + lax.axis_index("c")
        sbase = wid * _SEQ
        pltpu.sync_copy(idx_hbm.at[:, pl.ds(sbase, _SEQ)], idx_v)

        def gather(t, start):
            bb = t % _NBUF
            cp = pltpu.make_async_copy(
                table_hbm.at[idx_v.at[t]], rows_v.at[bb], gsem.at[bb]
            )
            cp.start() if start else cp.wait()

        def store(t, start):
            bb = t % _NBUF
            cp = pltpu.make_async_copy(
                rows_v.at[bb], out_hbm.at[t, pl.ds(sbase, _SEQ)], ssem.at[bb]
            )
            cp.start() if start else cp.wait()

        for t in range(_INFLIGHT):
            gather(t, start=True)

        def step(t, carry):
            gather(t, start=False)
            @pl.when(t >= _NBUF - _INFLIGHT)
            def _():
                store(t - (_NBUF - _INFLIGHT), start=False)

            store(t, start=True)

            @pl.when(t + _INFLIGHT < l)
            def _():
                gather(t + _INFLIGHT, start=True)

            return carry

        lax.fori_loop(0, l, step, 0)
        for t in range(max(0, l - (_NBUF - _INFLIGHT)), l):
            store(t, start=False)

    return emb


def kernel(token_idxs, table):
    b, l = token_idxs.shape
    v, d = table.shape
    idx_t = token_idxs.T.astype(jnp.int32)
    out = _build(b, l, d)(idx_t, table)
    return jnp.transpose(out, (1, 0, 2)).reshape(1, b, l, d)


# speedup vs baseline: 10.7842x; 1.0050x over previous
import functools

import jax
import jax.numpy as jnp
from jax import lax
from jax.experimental import pallas as pl
from jax.experimental.pallas import tpu as pltpu
from jax.experimental.pallas import tpu_sc as plsc

_NC = 2
_NS = 16
_NW = _NC * _NS
_SEQ = 128
_NBUF = 4
_INFLIGHT = 3


@functools.lru_cache(maxsize=None)
def _build(b, l, d):
    mesh = plsc.VectorSubcoreMesh(core_axis_name="c", subcore_axis_name="s")

    @functools.partial(
        pl.kernel,
        mesh=mesh,
        out_type=jax.ShapeDtypeStruct((l, b, d), jnp.float32),
        scratch_types=[
            pltpu.VMEM((l, _SEQ), jnp.int32),
            pltpu.VMEM((_NBUF, _SEQ, d), jnp.float32),
            pltpu.SemaphoreType.DMA((_NBUF,)),
            pltpu.SemaphoreType.DMA((_NBUF,)),
        ],
    )
    def emb(idx_hbm, table_hbm, out_hbm, idx_v, rows_v, gsem, ssem):
        wid = lax.axis_index("s") * _NC + lax.axis_index("c")
        sbase = wid * _SEQ
        pltpu.sync_copy(idx_hbm.at[:, pl.ds(sbase, _SEQ)], idx_v)

        def gather(t, start):
            bb = t % _NBUF
            cp = pltpu.make_async_copy(
                table_hbm.at[idx_v.at[t]], rows_v.at[bb], gsem.at[bb]
            )
            cp.start() if start else cp.wait()

        def store(t, start):
            bb = t % _NBUF
            cp = pltpu.make_async_copy(
                rows_v.at[bb], out_hbm.at[t, pl.ds(sbase, _SEQ)], ssem.at[bb]
            )
            cp.start() if start else cp.wait()

        for t in range(_INFLIGHT):
            gather(t, start=True)

        def step(t, carry):
            gather(t, start=False)
            @pl.when(t >= _NBUF - _INFLIGHT)
            def _():
                store(t - (_NBUF - _INFLIGHT), start=False)

            store(t, start=True)

            @pl.when(t + _INFLIGHT < l)
            def _():
                gather(t + _INFLIGHT, start=True)

            return carry

        lax.fori_loop(0, l, step, 0)
        for t in range(max(0, l - (_NBUF - _INFLIGHT)), l):
            store(t, start=False)

    return emb


def kernel(token_idxs, table):
    b, l = token_idxs.shape
    v, d = table.shape
    idx_t = token_idxs.T.astype(jnp.int32)
    out = _build(b, l, d)(idx_t, table)
    return jnp.transpose(out, (1, 0, 2)).reshape(1, b, l, d)


# speedup vs baseline: 10.8090x; 1.0023x over previous
import functools

import jax
import jax.numpy as jnp
from jax import lax
from jax.experimental import pallas as pl
from jax.experimental.pallas import tpu as pltpu
from jax.experimental.pallas import tpu_sc as plsc

_NC = 2
_NS = 16
_NW = _NC * _NS
_SEQ = 128
_NBUF = 6
_INFLIGHT = 4


@functools.lru_cache(maxsize=None)
def _build(b, l, d):
    mesh = plsc.VectorSubcoreMesh(core_axis_name="c", subcore_axis_name="s")

    @functools.partial(
        pl.kernel,
        mesh=mesh,
        out_type=jax.ShapeDtypeStruct((l, b, d), jnp.float32),
        scratch_types=[
            pltpu.VMEM((l, _SEQ), jnp.int32),
            pltpu.VMEM((_NBUF, _SEQ, d), jnp.float32),
            pltpu.SemaphoreType.DMA((_NBUF,)),
            pltpu.SemaphoreType.DMA((_NBUF,)),
        ],
    )
    def emb(idx_hbm, table_hbm, out_hbm, idx_v, rows_v, gsem, ssem):
        wid = lax.axis_index("s") * _NC + lax.axis_index("c")
        sbase = wid * _SEQ
        pltpu.sync_copy(idx_hbm.at[:, pl.ds(sbase, _SEQ)], idx_v)

        def gather(t, start):
            bb = t % _NBUF
            cp = pltpu.make_async_copy(
                table_hbm.at[idx_v.at[t]], rows_v.at[bb], gsem.at[bb]
            )
            cp.start() if start else cp.wait()

        def store(t, start):
            bb = t % _NBUF
            cp = pltpu.make_async_copy(
                rows_v.at[bb], out_hbm.at[t, pl.ds(sbase, _SEQ)], ssem.at[bb]
            )
            cp.start() if start else cp.wait()

        for t in range(_INFLIGHT):
            gather(t, start=True)

        def step(t, carry):
            gather(t, start=False)
            @pl.when(t >= _NBUF - _INFLIGHT)
            def _():
                store(t - (_NBUF - _INFLIGHT), start=False)

            store(t, start=True)

            @pl.when(t + _INFLIGHT < l)
            def _():
                gather(t + _INFLIGHT, start=True)

            return carry

        lax.fori_loop(0, l, step, 0)
        for t in range(max(0, l - (_NBUF - _INFLIGHT)), l):
            store(t, start=False)

    return emb


def kernel(token_idxs, table):
    b, l = token_idxs.shape
    v, d = table.shape
    idx_t = token_idxs.T.astype(jnp.int32)
    out = _build(b, l, d)(idx_t, table)
    return jnp.transpose(out, (1, 0, 2)).reshape(1, b, l, d)


# speedup vs baseline: 10.8952x; 1.0080x over previous
import functools

import jax
import jax.numpy as jnp
from jax import lax
from jax.experimental import pallas as pl
from jax.experimental.pallas import tpu as pltpu
from jax.experimental.pallas import tpu_sc as plsc

_NC = 2
_NS = 16
_NW = _NC * _NS
_SEQ = 128
_NBUF = 7
_INFLIGHT = 5


@functools.lru_cache(maxsize=None)
def _build(b, l, d):
    mesh = plsc.VectorSubcoreMesh(core_axis_name="c", subcore_axis_name="s")

    @functools.partial(
        pl.kernel,
        mesh=mesh,
        out_type=jax.ShapeDtypeStruct((l, b, d), jnp.float32),
        scratch_types=[
            pltpu.VMEM((l, _SEQ), jnp.int32),
            pltpu.VMEM((_NBUF, _SEQ, d), jnp.float32),
            pltpu.SemaphoreType.DMA((_NBUF,)),
            pltpu.SemaphoreType.DMA((_NBUF,)),
        ],
    )
    def emb(idx_hbm, table_hbm, out_hbm, idx_v, rows_v, gsem, ssem):
        wid = lax.axis_index("s") * _NC + lax.axis_index("c")
        sbase = wid * _SEQ
        pltpu.sync_copy(idx_hbm.at[:, pl.ds(sbase, _SEQ)], idx_v)

        def gather(t, start):
            bb = t % _NBUF
            cp = pltpu.make_async_copy(
                table_hbm.at[idx_v.at[t]], rows_v.at[bb], gsem.at[bb]
            )
            cp.start() if start else cp.wait()

        def store(t, start):
            bb = t % _NBUF
            cp = pltpu.make_async_copy(
                rows_v.at[bb], out_hbm.at[t, pl.ds(sbase, _SEQ)], ssem.at[bb]
            )
            cp.start() if start else cp.wait()

        for t in range(_INFLIGHT):
            gather(t, start=True)

        def step(t, carry):
            gather(t, start=False)
            @pl.when(t >= _NBUF - _INFLIGHT)
            def _():
                store(t - (_NBUF - _INFLIGHT), start=False)

            store(t, start=True)

            @pl.when(t + _INFLIGHT < l)
            def _():
                gather(t + _INFLIGHT, start=True)

            return carry

        lax.fori_loop(0, l, step, 0)
        for t in range(max(0, l - (_NBUF - _INFLIGHT)), l):
            store(t, start=False)

    return emb


def kernel(token_idxs, table):
    b, l = token_idxs.shape
    v, d = table.shape
    idx_t = token_idxs.T.astype(jnp.int32)
    out = _build(b, l, d)(idx_t, table)
    return jnp.transpose(out, (1, 0, 2)).reshape(1, b, l, d)
